# baseline (speedup 1.0000x reference)
import functools

import jax
import jax.numpy as jnp
from jax import lax
from jax.experimental import pallas as pl
from jax.experimental.pallas import tpu as pltpu
from jax.experimental.pallas import tpu_sc as plsc

BATCH = 16384
SEQ = 50
DIM = 32
N_ROWS = BATCH * SEQ
OUT_FLAT = BATCH * DIM * SEQ
NC, NS = 2, 16
NW = NC * NS
ELEMS_W = BATCH // NW
CB = 16
N_CHUNKS = ELEMS_W // CB
ROWS_C = CB * SEQ
IDX_W = 100
SUB = ROWS_C // IDX_W
EL_F = DIM * SEQ


def _body(idx_hbm, table_hbm, out_hbm, idx_v, rows_v, obuf, sem):
    wid = lax.axis_index("s") * NC + lax.axis_index("c")
    iota50 = lax.iota(jnp.int32, 16) * SEQ

    def chunk(c, _):
        e0 = wid * ELEMS_W + c * CB
        idx_off = pl.multiple_of(e0 * SEQ // IDX_W, 8)
        pltpu.sync_copy(idx_hbm.at[pl.ds(idx_off, SUB)], idx_v)
        copies = [
            pltpu.async_copy(
                table_hbm.at[idx_v.at[j]],
                rows_v.at[pl.ds(j * IDX_W, IDX_W)],
                sem,
            )
            for j in range(SUB)
        ]
        for cp in copies:
            cp.wait()

        def elem(e, _):
            base = e * EL_F
            r0 = e * SEQ
            for s in range(SEQ):
                v0 = rows_v[r0 + s, pl.ds(0, 16)]
                v1 = rows_v[r0 + s, pl.ds(16, 16)]
                i0 = iota50 + (base + s)
                plsc.store_scatter(obuf, [i0], v0)
                plsc.store_scatter(obuf, [i0 + (16 * SEQ)], v1)
            return 0

        lax.fori_loop(0, CB, elem, 0)
        out_off = pl.multiple_of(e0 * EL_F, 8)
        pltpu.sync_copy(obuf, out_hbm.at[pl.ds(out_off, CB * EL_F)])
        return 0

    lax.fori_loop(0, N_CHUNKS, chunk, 0)


_fused = functools.partial(
    pl.kernel,
    mesh=plsc.VectorSubcoreMesh(core_axis_name="c", subcore_axis_name="s"),
    out_type=jax.ShapeDtypeStruct((OUT_FLAT,), jnp.float32),
    scratch_types=[
        pltpu.VMEM((SUB, IDX_W), jnp.int32),
        pltpu.VMEM((ROWS_C, DIM), jnp.float32),
        pltpu.VMEM((CB * EL_F,), jnp.float32),
        pltpu.SemaphoreType.DMA,
    ],
    compiler_params=pltpu.CompilerParams(
        use_tc_tiling_on_sc=False, needs_layout_passes=False
    ),
)(_body)


def kernel(inputs, z):
    idx2d = jnp.reshape(inputs, (N_ROWS // IDX_W, IDX_W)).astype(jnp.int32)
    flat = _fused(idx2d, z)
    return jnp.reshape(flat, (BATCH, DIM, SEQ))


# speedup vs baseline: 1.0692x; 1.0692x over previous
import functools

import jax
import jax.numpy as jnp
from jax import lax
from jax.experimental import pallas as pl
from jax.experimental.pallas import tpu as pltpu
from jax.experimental.pallas import tpu_sc as plsc

BATCH = 16384
SEQ = 50
DIM = 32
NC, NS = 2, 16
NW = NC * NS
ELEMS_W = BATCH // NW
CB = 16
N_CHUNKS = ELEMS_W // CB
ROWS_C = CB * SEQ


def _body(idx_hbm, table_hbm, out_hbm, idx_v, rows_v, obuf, sem):
    wid = lax.axis_index("s") * NC + lax.axis_index("c")
    iota16 = lax.iota(jnp.int32, 16)

    def chunk(c, _):
        e0 = wid * ELEMS_W + c * CB
        pltpu.sync_copy(idx_hbm.at[pl.ds(e0, CB)], idx_v)
        copies = [
            pltpu.async_copy(
                table_hbm.at[idx_v.at[e]],
                rows_v.at[pl.ds(e * SEQ, SEQ)],
                sem,
            )
            for e in range(CB)
        ]
        for cp in copies:
            cp.wait()

        def elem(e, _):
            r0 = e * SEQ
            row0 = iota16 + e * DIM
            row1 = row0 + 16
            for s in range(SEQ):
                s_vec = jnp.full((16,), s, jnp.int32)
                v0 = rows_v[r0 + s, pl.ds(0, 16)]
                v1 = rows_v[r0 + s, pl.ds(16, 16)]
                plsc.store_scatter(obuf, [row0, s_vec], v0)
                plsc.store_scatter(obuf, [row1, s_vec], v1)
            return 0

        lax.fori_loop(0, CB, elem, 0)
        out_off = pl.multiple_of(e0 * DIM, 8)
        pltpu.sync_copy(obuf, out_hbm.at[pl.ds(out_off, CB * DIM)])
        return 0

    lax.fori_loop(0, N_CHUNKS, chunk, 0)


_fused = functools.partial(
    pl.kernel,
    mesh=plsc.VectorSubcoreMesh(core_axis_name="c", subcore_axis_name="s"),
    out_type=jax.ShapeDtypeStruct((BATCH * DIM, SEQ), jnp.float32),
    scratch_types=[
        pltpu.VMEM((CB, SEQ), jnp.int32),
        pltpu.VMEM((ROWS_C, DIM), jnp.float32),
        pltpu.VMEM((CB * DIM, SEQ), jnp.float32),
        pltpu.SemaphoreType.DMA,
    ],
    compiler_params=pltpu.CompilerParams(
        use_tc_tiling_on_sc=False, needs_layout_passes=False
    ),
)(_body)


def kernel(inputs, z):
    out2d = _fused(inputs.astype(jnp.int32), z)
    return jnp.reshape(out2d, (BATCH, DIM, SEQ))
